# baseline (speedup 1.0000x reference)
import functools

import jax
import jax.numpy as jnp
from jax import lax
from jax.experimental import pallas as pl
from jax.experimental.pallas import tpu as pltpu
from jax.experimental.pallas import tpu_sc as plsc

NUM_COLLISIONS = 4
D = 64
L16 = 16
C = 512
G = 128
NG = C // G


def _sc_body(total_rows, num_cores, idx_hbm, wq_hbm, wr_hbm, out_hbm,
             idxbuf, qidx_a, rbuf_a, qrows_a, qidx_b, rbuf_b, qrows_b,
             wr_v, sem_a, sem_b):
    wid = lax.axis_index("s") * num_cores + lax.axis_index("c")
    rows_per_w = total_rows // (num_cores * 16)
    nchunks = rows_per_w // C
    base = wid * rows_per_w

    pltpu.sync_copy(wr_hbm, wr_v)

    cols = [lax.iota(jnp.int32, L16) + dj * L16 for dj in range(D // L16)]

    bufs = ((qidx_a, rbuf_a, qrows_a, sem_a),
            (qidx_b, rbuf_b, qrows_b, sem_b))

    def load_qr(c, s):
        qidx, rbuf, _, _ = bufs[s]
        pltpu.sync_copy(idx_hbm.at[pl.ds(base + c * C, C)], idxbuf)
        for j in range(NG):
            def qr(i, _, j=j):
                v = idxbuf[pl.ds(j * G + i * L16, L16)]
                qidx[j, pl.ds(i * L16, L16)] = v >> 2
                rbuf[pl.ds(j * G + i * L16, L16)] = v & (NUM_COLLISIONS - 1)
                return 0
            lax.fori_loop(0, G // L16, qr, 0)

    def gathers(s):
        qidx, _, qrows, sem = bufs[s]
        return [pltpu.make_async_copy(
            wq_hbm.at[qidx.at[j]], qrows.at[pl.ds(j * G, G)], sem)
            for j in range(NG)]

    def issue(s):
        for cp in gathers(s):
            cp.start()

    def combine_out(c, s):
        _, rbuf, qrows, _ = bufs[s]
        for cp in gathers(s):
            cp.wait()

        def comb(t, _):
            base_row = t * 4
            for u in range(4):
                row = base_row + u
                r16 = plsc.load_gather(rbuf, [jnp.full((L16,), row,
                                                       jnp.int32)])
                for dj in range(D // L16):
                    mult = plsc.load_gather(wr_v, [r16, cols[dj]])
                    sl = pl.ds(dj * L16, L16)
                    qrows[row, sl] = qrows[row, sl] * mult
            return 0
        lax.fori_loop(0, C // 4, comb, 0)
        pltpu.sync_copy(qrows, out_hbm.at[pl.ds(base + c * C, C)])

    load_qr(0, 0)
    issue(0)

    def pair(c2, carry):
        c = 2 * c2
        load_qr(c + 1, 1)
        issue(1)
        combine_out(c, 0)
        load_qr(c + 2, 0)
        issue(0)
        combine_out(c + 1, 1)
        return carry

    lax.fori_loop(0, nchunks // 2 - 1, pair, 0)

    c_last = nchunks - 2
    load_qr(c_last + 1, 1)
    issue(1)
    combine_out(c_last, 0)
    combine_out(c_last + 1, 1)


def kernel(input, W_q, W_r):
    B, L = input.shape
    total = B * L
    idx_flat = input.reshape(total).astype(jnp.int32)

    info = plsc.get_sparse_core_info()
    nc = info.num_cores

    mesh = plsc.VectorSubcoreMesh(core_axis_name="c", subcore_axis_name="s")
    out_flat = pl.kernel(
        functools.partial(_sc_body, total, nc),
        out_type=jax.ShapeDtypeStruct((total, D), jnp.float32),
        mesh=mesh,
        scratch_types=[
            pltpu.VMEM((C,), jnp.int32),
            pltpu.VMEM((NG, G), jnp.int32),
            pltpu.VMEM((C,), jnp.int32),
            pltpu.VMEM((C, D), jnp.float32),
            pltpu.VMEM((NG, G), jnp.int32),
            pltpu.VMEM((C,), jnp.int32),
            pltpu.VMEM((C, D), jnp.float32),
            pltpu.VMEM((NUM_COLLISIONS, D), jnp.float32),
            pltpu.SemaphoreType.DMA,
            pltpu.SemaphoreType.DMA,
        ],
        compiler_params=pltpu.CompilerParams(use_tc_tiling_on_sc=False,
                                             needs_layout_passes=False),
    )(idx_flat, W_q, W_r[:NUM_COLLISIONS])

    return out_flat.reshape(B, L, D)


# speedup vs baseline: 1.3931x; 1.3931x over previous
import functools

import jax
import jax.numpy as jnp
from jax import lax
from jax.experimental import pallas as pl
from jax.experimental.pallas import tpu as pltpu
from jax.experimental.pallas import tpu_sc as plsc

NUM_COLLISIONS = 4
D = 64
L16 = 16

CQ = 125
CT = CQ * NUM_COLLISIONS
NCHUNKS1 = 2000

C = 512
G = 128
NG = C // G

_PARAMS = pltpu.CompilerParams(use_tc_tiling_on_sc=False,
                               needs_layout_passes=False)


def _build_body(num_workers, wq_hbm, wr_hbm, t_hbm,
                wq_a, wq_b, t_a, t_b, wr_v, isem_a, isem_b, osem_a, osem_b):
    nc = num_workers // 16
    wid = lax.axis_index("s") * nc + lax.axis_index("c")

    pltpu.sync_copy(wr_hbm, wr_v)
    wrv = [[wr_v[r, pl.ds(dj * L16, L16)] for dj in range(D // L16)]
           for r in range(NUM_COLLISIONS)]

    bufs = ((wq_a, t_a, isem_a, osem_a), (wq_b, t_b, isem_b, osem_b))
    last = NCHUNKS1 - 1

    def g_of(k):
        return jnp.minimum(wid + num_workers * k, last)

    def start_in(k, s):
        wq, _, isem, _ = bufs[s]
        pltpu.make_async_copy(
            wq_hbm.at[pl.ds(g_of(k) * CQ, CQ)], wq, isem).start()

    def wait_in(s):
        wq, _, isem, _ = bufs[s]
        pltpu.make_async_copy(wq_hbm.at[pl.ds(0, CQ)], wq, isem).wait()

    def compute(s):
        wq, tb, _, _ = bufs[s]

        def row(i, _):
            for dj in range(D // L16):
                sl = pl.ds(dj * L16, L16)
                w = wq[i, sl]
                for r in range(NUM_COLLISIONS):
                    tb[i * NUM_COLLISIONS + r, sl] = w * wrv[r][dj]
            return 0
        lax.fori_loop(0, CQ, row, 0)

    def start_out(k, s):
        _, tb, _, osem = bufs[s]
        pltpu.make_async_copy(
            tb, t_hbm.at[pl.ds(g_of(k) * CT, CT)], osem).start()

    def wait_out(s):
        _, tb, _, osem = bufs[s]
        pltpu.make_async_copy(tb, t_hbm.at[pl.ds(0, CT)], osem).wait()

    start_in(0, 0)
    start_in(1, 1)
    wait_in(0)
    compute(0)
    start_out(0, 0)
    start_in(2, 0)
    wait_in(1)
    compute(1)
    start_out(1, 1)

    def pair(p, carry):
        k = 2 * p
        start_in(k + 1, 1)
        wait_out(0)
        wait_in(0)
        compute(0)
        start_out(k, 0)
        start_in(k + 2, 0)
        wait_out(1)
        wait_in(1)
        compute(1)
        start_out(k + 1, 1)
        return carry

    lax.fori_loop(1, 31, pair, 0)

    wait_in(0)
    wait_out(0)

    def tail():
        compute(0)
        start_out(62, 0)
        wait_out(0)

    lax.cond(wid < NCHUNKS1 - 62 * num_workers, tail, lambda: None)
    wait_out(1)


def _lookup_body(total_rows, num_workers, idx_hbm, t_hbm, out_hbm,
                 idx_a, idx_b, rows_a, rows_b, sem_a, sem_b):
    nc = num_workers // 16
    wid = lax.axis_index("s") * nc + lax.axis_index("c")
    rows_per_w = total_rows // num_workers
    nchunks = rows_per_w // C
    base = wid * rows_per_w
    gbase = base // G

    bufs = ((idx_a, rows_a, sem_a), (idx_b, rows_b, sem_b))

    def load_idx(c, s):
        idxb, _, _ = bufs[s]
        pltpu.sync_copy(idx_hbm.at[pl.ds(gbase + c * NG, NG)], idxb)

    def gathers(s):
        idxb, rows, sem = bufs[s]
        return [pltpu.make_async_copy(
            t_hbm.at[idxb.at[j]], rows.at[pl.ds(j * G, G)], sem)
            for j in range(NG)]

    def issue(s):
        for cp in gathers(s):
            cp.start()

    def drain_out(c, s):
        _, rows, _ = bufs[s]
        for cp in gathers(s):
            cp.wait()
        pltpu.sync_copy(rows, out_hbm.at[pl.ds(base + c * C, C)])

    load_idx(0, 0)
    issue(0)

    def pair(c2, carry):
        c = 2 * c2
        load_idx(c + 1, 1)
        issue(1)
        drain_out(c, 0)
        load_idx(c + 2, 0)
        issue(0)
        drain_out(c + 1, 1)
        return carry

    lax.fori_loop(0, nchunks // 2 - 1, pair, 0)

    c_last = nchunks - 2
    load_idx(c_last + 1, 1)
    issue(1)
    drain_out(c_last, 0)
    drain_out(c_last + 1, 1)


def kernel(input, W_q, W_r):
    B, L = input.shape
    total = B * L
    idx2d = input.reshape(total // G, G).astype(jnp.int32)

    info = plsc.get_sparse_core_info()
    nw = info.num_cores * info.num_subcores
    mesh = plsc.VectorSubcoreMesh(core_axis_name="c", subcore_axis_name="s")

    table = pl.kernel(
        functools.partial(_build_body, nw),
        out_type=jax.ShapeDtypeStruct((W_q.shape[0] * NUM_COLLISIONS, D),
                                      jnp.float32),
        mesh=mesh,
        scratch_types=[
            pltpu.VMEM((CQ, D), jnp.float32),
            pltpu.VMEM((CQ, D), jnp.float32),
            pltpu.VMEM((CT, D), jnp.float32),
            pltpu.VMEM((CT, D), jnp.float32),
            pltpu.VMEM((NUM_COLLISIONS, D), jnp.float32),
            pltpu.SemaphoreType.DMA,
            pltpu.SemaphoreType.DMA,
            pltpu.SemaphoreType.DMA,
            pltpu.SemaphoreType.DMA,
        ],
        compiler_params=_PARAMS,
    )(W_q, W_r[:NUM_COLLISIONS])

    out_flat = pl.kernel(
        functools.partial(_lookup_body, total, nw),
        out_type=jax.ShapeDtypeStruct((total, D), jnp.float32),
        mesh=mesh,
        scratch_types=[
            pltpu.VMEM((NG, G), jnp.int32),
            pltpu.VMEM((NG, G), jnp.int32),
            pltpu.VMEM((C, D), jnp.float32),
            pltpu.VMEM((C, D), jnp.float32),
            pltpu.SemaphoreType.DMA,
            pltpu.SemaphoreType.DMA,
        ],
        compiler_params=_PARAMS,
    )(idx2d, table)

    return out_flat.reshape(B, L, D)
